# baseline (device time: 25757 ns/iter reference)
import jax
import jax.numpy as jnp
from jax import lax
from jax.experimental import pallas as pl
from jax.experimental.pallas import tpu as pltpu

N_DEV = 32
M = 512
N = 512
ROWS = M // N_DEV
NH = 2
NC = N // NH


def kernel(A, B):
    def body(a_ref, b_ref, out_ref, z_ref, red_ref, recv_ref,
             send_sems1, recv_sems1, send_sems2, recv_sems2, copy_sem):
        my_id = lax.axis_index("i")

        barrier_sem = pltpu.get_barrier_semaphore()
        for off in range(1, N_DEV):
            peer = (my_id + off) % N_DEV
            pl.semaphore_signal(
                barrier_sem, inc=1,
                device_id=(peer,), device_id_type=pl.DeviceIdType.MESH,
            )

        z = jnp.dot(
            a_ref[...].astype(jnp.bfloat16),
            b_ref[...].astype(jnp.bfloat16),
            preferred_element_type=jnp.float32,
        )
        z_ref[...] = z.astype(jnp.bfloat16)

        pl.semaphore_wait(barrier_sem, N_DEV - 1)

        ph1 = [[] for _ in range(NH)]
        own = []
        for h in range(NH):
            for off in range(1, N_DEV):
                peer = (my_id + off) % N_DEV
                rdma = pltpu.make_async_remote_copy(
                    src_ref=z_ref.at[pl.ds(peer * ROWS, ROWS),
                                     pl.ds(h * NC, NC)],
                    dst_ref=recv_ref.at[h, N_DEV - off],
                    send_sem=send_sems1.at[h],
                    recv_sem=recv_sems1.at[h],
                    device_id=(peer,),
                    device_id_type=pl.DeviceIdType.MESH,
                )
                rdma.start()
                ph1[h].append(rdma)
            cp = pltpu.make_async_copy(
                z_ref.at[pl.ds(my_id * ROWS, ROWS), pl.ds(h * NC, NC)],
                recv_ref.at[h, 0],
                copy_sem,
            )
            cp.start()
            own.append(cp)

        ph2 = [[] for _ in range(NH)]
        mine = []
        for h in range(NH):
            own[h].wait()
            for rdma in ph1[h]:
                rdma.wait_recv()

            acc = recv_ref[h, 0].astype(jnp.float32)
            for s in range(1, N_DEV):
                acc += recv_ref[h, s].astype(jnp.float32)
            g = 0.5 * acc * (
                1.0 + jnp.tanh(
                    0.7978845608 * (acc + 0.044715 * acc * acc * acc)
                )
            )
            red_ref[:, pl.ds(h * NC, NC)] = g.astype(jnp.bfloat16)

            cp = pltpu.make_async_copy(
                red_ref.at[:, pl.ds(h * NC, NC)],
                out_ref.at[pl.ds(my_id * ROWS, ROWS), pl.ds(h * NC, NC)],
                copy_sem,
            )
            cp.start()
            mine.append(cp)

            for off in range(1, N_DEV):
                peer = (my_id + off) % N_DEV
                rdma = pltpu.make_async_remote_copy(
                    src_ref=red_ref.at[:, pl.ds(h * NC, NC)],
                    dst_ref=out_ref.at[pl.ds(my_id * ROWS, ROWS),
                                       pl.ds(h * NC, NC)],
                    send_sem=send_sems2.at[h],
                    recv_sem=recv_sems2.at[h],
                    device_id=(peer,),
                    device_id_type=pl.DeviceIdType.MESH,
                )
                rdma.start()
                ph2[h].append(rdma)

        for h in range(NH):
            mine[h].wait()
            for rdma in ph2[h]:
                rdma.wait_recv()
        for h in range(NH):
            for rdma in ph1[h]:
                rdma.wait_send()
            for rdma in ph2[h]:
                rdma.wait_send()

    return pl.pallas_call(
        body,
        out_shape=jax.ShapeDtypeStruct((M, N), jnp.bfloat16),
        in_specs=[
            pl.BlockSpec(memory_space=pltpu.VMEM),
            pl.BlockSpec(memory_space=pltpu.VMEM),
        ],
        out_specs=pl.BlockSpec(memory_space=pltpu.VMEM),
        scratch_shapes=[
            pltpu.VMEM((M, N), jnp.bfloat16),
            pltpu.VMEM((ROWS, N), jnp.bfloat16),
            pltpu.VMEM((NH, N_DEV, ROWS, NC), jnp.bfloat16),
            pltpu.SemaphoreType.DMA((NH,)),
            pltpu.SemaphoreType.DMA((NH,)),
            pltpu.SemaphoreType.DMA((NH,)),
            pltpu.SemaphoreType.DMA((NH,)),
            pltpu.SemaphoreType.DMA,
        ],
        compiler_params=pltpu.CompilerParams(collective_id=0),
    )(A, B)


# device time: 23286 ns/iter; 1.1061x vs baseline; 1.1061x over previous
import jax
import jax.numpy as jnp
from jax import lax
from jax.experimental import pallas as pl
from jax.experimental.pallas import tpu as pltpu

N_DEV = 32
M = 512
N = 512
ROWS = M // N_DEV
NH = 2
NC = N // NH


def kernel(A, B):
    far_first = sorted(range(1, N_DEV), key=lambda o: -min(o, N_DEV - o))

    def body(a_ref, b_ref, out_ref, z_ref, red_ref, recv_ref,
             send_sems1, recv_sems1, send_sems2, recv_sems2, copy_sems):
        my_id = lax.axis_index("i")

        barrier_sem = pltpu.get_barrier_semaphore()
        for off in range(1, N_DEV):
            peer = (my_id + off) % N_DEV
            pl.semaphore_signal(
                barrier_sem, inc=1,
                device_id=(peer,), device_id_type=pl.DeviceIdType.MESH,
            )

        z = jnp.dot(
            a_ref[...].astype(jnp.bfloat16),
            b_ref[...].astype(jnp.bfloat16),
            preferred_element_type=jnp.float32,
        )
        z_ref[...] = z.astype(jnp.bfloat16)

        pl.semaphore_wait(barrier_sem, N_DEV - 1)

        ph1 = [[] for _ in range(NH)]
        own = []
        for h in range(NH):
            for off in far_first:
                peer = (my_id + off) % N_DEV
                rdma = pltpu.make_async_remote_copy(
                    src_ref=z_ref.at[pl.ds(peer * ROWS, ROWS),
                                     pl.ds(h * NC, NC)],
                    dst_ref=recv_ref.at[h, N_DEV - off],
                    send_sem=send_sems1.at[h],
                    recv_sem=recv_sems1.at[h],
                    device_id=(peer,),
                    device_id_type=pl.DeviceIdType.MESH,
                )
                rdma.start()
                ph1[h].append(rdma)
            cp = pltpu.make_async_copy(
                z_ref.at[pl.ds(my_id * ROWS, ROWS), pl.ds(h * NC, NC)],
                recv_ref.at[h, 0],
                copy_sems.at[h],
            )
            cp.start()
            own.append(cp)

        ph2 = [[] for _ in range(NH)]
        mine = []
        for h in range(NH):
            own[h].wait()
            for rdma in ph1[h]:
                rdma.wait_recv()

            acc = recv_ref[h, 0].astype(jnp.float32)
            for s in range(1, N_DEV):
                acc += recv_ref[h, s].astype(jnp.float32)
            g = 0.5 * acc * (
                1.0 + jnp.tanh(
                    0.7978845608 * (acc + 0.044715 * acc * acc * acc)
                )
            )
            red_ref[:, pl.ds(h * NC, NC)] = g.astype(jnp.bfloat16)

            cp = pltpu.make_async_copy(
                red_ref.at[:, pl.ds(h * NC, NC)],
                out_ref.at[pl.ds(my_id * ROWS, ROWS), pl.ds(h * NC, NC)],
                copy_sems.at[NH + h],
            )
            cp.start()
            mine.append(cp)

            for off in far_first:
                peer = (my_id + off) % N_DEV
                rdma = pltpu.make_async_remote_copy(
                    src_ref=red_ref.at[:, pl.ds(h * NC, NC)],
                    dst_ref=out_ref.at[pl.ds(my_id * ROWS, ROWS),
                                       pl.ds(h * NC, NC)],
                    send_sem=send_sems2.at[h],
                    recv_sem=recv_sems2.at[h],
                    device_id=(peer,),
                    device_id_type=pl.DeviceIdType.MESH,
                )
                rdma.start()
                ph2[h].append(rdma)

        for h in range(NH):
            mine[h].wait()
            for rdma in ph2[h]:
                rdma.wait_recv()
        for h in range(NH):
            for rdma in ph1[h]:
                rdma.wait_send()
            for rdma in ph2[h]:
                rdma.wait_send()

    return pl.pallas_call(
        body,
        out_shape=jax.ShapeDtypeStruct((M, N), jnp.bfloat16),
        in_specs=[
            pl.BlockSpec(memory_space=pltpu.VMEM),
            pl.BlockSpec(memory_space=pltpu.VMEM),
        ],
        out_specs=pl.BlockSpec(memory_space=pltpu.VMEM),
        scratch_shapes=[
            pltpu.VMEM((M, N), jnp.bfloat16),
            pltpu.VMEM((ROWS, N), jnp.bfloat16),
            pltpu.VMEM((NH, N_DEV, ROWS, NC), jnp.bfloat16),
            pltpu.SemaphoreType.DMA((NH,)),
            pltpu.SemaphoreType.DMA((NH,)),
            pltpu.SemaphoreType.DMA((NH,)),
            pltpu.SemaphoreType.DMA((NH,)),
            pltpu.SemaphoreType.DMA((2 * NH,)),
        ],
        compiler_params=pltpu.CompilerParams(collective_id=0),
    )(A, B)
